# baseline (speedup 1.0000x reference)
import functools

import jax
import jax.numpy as jnp
from jax import lax
from jax.experimental import pallas as pl
from jax.experimental.pallas import tpu as pltpu
from jax.experimental.pallas import tpu_sc as plsc

_B, _C, _Y, _X = 8, 64, 224, 224
_FC = 4
_NBC = _B * _FC
_OY, _OX = 32, 32


def _stats_body(fm_ref, gxr_ref, gyc_ref, g32r_ref, g32c_ref,
                mx_ref, my_ref, sc_ref, rid_ref, xu_ref, xl_ref,
                wuu_ref, wul_ref, wlu_ref, wll_ref):
    bc = pl.program_id(0)
    f = fm_ref[0]
    m = jnp.max(f)
    e = jnp.exp(f - m)
    col = jnp.sum(e, axis=0, keepdims=True)
    row = jnp.sum(e, axis=1, keepdims=True)
    s_tot = jnp.sum(col)
    gx = gxr_ref[...]
    gy = gyc_ref[...]
    mean_x = jnp.sum(col * gx) / s_tot
    mean_y = jnp.sum(row * gy) / s_tot
    scale = (jnp.sum(col * jnp.abs(gx - mean_x))
             + jnp.sum(row * jnp.abs(gy - mean_y))) / s_tot

    g32r = g32r_ref[...]
    g32c = g32c_ref[...]
    x_raw = ((g32r * scale + mean_x) + 1.0) * (_X / 2.0)
    xu = jnp.clip(jnp.ceil(x_raw), 0.0, _X - 1.0)
    xl = jnp.clip(jnp.floor(x_raw), 0.0, _X - 1.0)
    wxu = x_raw - xl
    wxl = xu - x_raw
    y_rawr = ((g32r * scale + mean_y) + 1.0) * (_Y / 2.0)
    yur = jnp.clip(jnp.ceil(y_rawr), 0.0, _Y - 1.0)
    ylr = jnp.clip(jnp.floor(y_rawr), 0.0, _Y - 1.0)
    y_rawc = ((g32c * scale + mean_y) + 1.0) * (_Y / 2.0)
    yuc = jnp.clip(jnp.ceil(y_rawc), 0.0, _Y - 1.0)
    ylc = jnp.clip(jnp.floor(y_rawc), 0.0, _Y - 1.0)
    wyu = y_rawc - ylc
    wyl = yuc - y_rawc

    mx_ref[...] = jnp.full((1, 1, 32), mean_x, jnp.float32)
    my_ref[...] = jnp.full((1, 1, 32), mean_y, jnp.float32)
    sc_ref[...] = jnp.full((1, 1, 32), scale, jnp.float32)

    b = bc // _FC
    ybase = jnp.concatenate([yur, ylr], axis=1).astype(jnp.int32)
    chv = lax.broadcasted_iota(jnp.int32, (_C, 1), 0)
    rid = (b * _C + chv) * _Y + ybase
    rid_ref[...] = rid.reshape(1, _C, 2 * _OY)

    xu_ref[...] = xu.astype(jnp.int32).reshape(1, 1, _OX)
    xl_ref[...] = xl.astype(jnp.int32).reshape(1, 1, _OX)
    wuu_ref[...] = (wyu * wxu).reshape(1, _OY, _OX)
    wul_ref[...] = (wyu * wxl).reshape(1, _OY, _OX)
    wlu_ref[...] = (wyl * wxu).reshape(1, _OY, _OX)
    wll_ref[...] = (wyl * wxl).reshape(1, _OY, _OX)


def _tc_stats(fm32, gxr, gyc, g32r, g32c):
    n = _NBC
    f32 = jnp.float32
    i32 = jnp.int32
    outs = [
        jax.ShapeDtypeStruct((n, 1, 32), f32),
        jax.ShapeDtypeStruct((n, 1, 32), f32),
        jax.ShapeDtypeStruct((n, 1, 32), f32),
        jax.ShapeDtypeStruct((n, _C, 2 * _OY), i32),
        jax.ShapeDtypeStruct((n, 1, _OX), i32),
        jax.ShapeDtypeStruct((n, 1, _OX), i32),
        jax.ShapeDtypeStruct((n, _OY, _OX), f32),
        jax.ShapeDtypeStruct((n, _OY, _OX), f32),
        jax.ShapeDtypeStruct((n, _OY, _OX), f32),
        jax.ShapeDtypeStruct((n, _OY, _OX), f32),
    ]
    full = lambda shp: pl.BlockSpec(shp, lambda i: (0, 0))
    blk = lambda shp: pl.BlockSpec(shp, lambda i: (i, 0, 0))
    return pl.pallas_call(
        _stats_body,
        grid=(n,),
        in_specs=[
            pl.BlockSpec((1, _Y, _X), lambda i: (i, 0, 0)),
            full((1, _X)), full((_Y, 1)), full((1, _OX)), full((_OY, 1)),
        ],
        out_specs=[
            blk((1, 1, 32)), blk((1, 1, 32)), blk((1, 1, 32)),
            blk((1, _C, 2 * _OY)), blk((1, 1, _OX)), blk((1, 1, _OX)),
            blk((1, _OY, _OX)), blk((1, _OY, _OX)),
            blk((1, _OY, _OX)), blk((1, _OY, _OX)),
        ],
        out_shape=outs,
    )(fm32, gxr, gyc, g32r, g32c)


def _sc_bilinear(imgrows, rid, xu, xl, wuu, wul, wlu, wll):
    mesh = plsc.VectorSubcoreMesh(core_axis_name="c", subcore_axis_name="s")

    @functools.partial(
        pl.kernel,
        mesh=mesh,
        compiler_params=pltpu.CompilerParams(use_tc_tiling_on_sc=False,
                                             needs_layout_passes=False),
        out_type=jax.ShapeDtypeStruct((_B * _C * _FC, _OY * _OX), jnp.float32),
        scratch_types=[
            pltpu.VMEM((_C, 2 * _OY), jnp.int32),
            pltpu.VMEM((2 * _OY, _X), jnp.float32),
            pltpu.VMEM((_OX,), jnp.int32),
            pltpu.VMEM((_OX,), jnp.int32),
            pltpu.VMEM((_OY, _OX), jnp.float32),
            pltpu.VMEM((_OY, _OX), jnp.float32),
            pltpu.VMEM((_OY, _OX), jnp.float32),
            pltpu.VMEM((_OY, _OX), jnp.float32),
            pltpu.VMEM((_OY * _OX,), jnp.float32),
            pltpu.SemaphoreType.DMA,
        ],
    )
    def body(img_hbm, rid_hbm, xu_hbm, xl_hbm, wuu_hbm, wul_hbm, wlu_hbm,
             wll_hbm, out_hbm, rid_v, rows_v, xu_v, xl_v, wuu_v, wul_v,
             wlu_v, wll_v, orow_v, sem):
        wid = lax.axis_index("s") * 2 + lax.axis_index("c")
        b = wid // _FC
        c = wid % _FC
        pltpu.sync_copy(rid_hbm.at[wid], rid_v)
        pltpu.sync_copy(xu_hbm.at[wid], xu_v)
        pltpu.sync_copy(xl_hbm.at[wid], xl_v)
        pltpu.sync_copy(wuu_hbm.at[wid], wuu_v)
        pltpu.sync_copy(wul_hbm.at[wid], wul_v)
        pltpu.sync_copy(wlu_hbm.at[wid], wlu_v)
        pltpu.sync_copy(wll_hbm.at[wid], wll_v)

        def ch_body(ch, carry):
            pltpu.async_copy(img_hbm.at[rid_v.at[ch]], rows_v, sem).wait()
            for i in range(_OY):
                for h in range(2):
                    sl = pl.ds(h * 16, 16)
                    xu16 = xu_v[sl]
                    xl16 = xl_v[sl]
                    yu16 = jnp.full((16,), i, jnp.int32)
                    yl16 = jnp.full((16,), _OY + i, jnp.int32)
                    puu = plsc.load_gather(rows_v, [yu16, xu16])
                    pul = plsc.load_gather(rows_v, [yu16, xl16])
                    plu = plsc.load_gather(rows_v, [yl16, xu16])
                    pll = plsc.load_gather(rows_v, [yl16, xl16])
                    o = (wuu_v[i, sl] * puu + wul_v[i, sl] * pul
                         + wlu_v[i, sl] * plu + wll_v[i, sl] * pll)
                    orow_v[pl.ds(i * _OX + h * 16, 16)] = o
            pltpu.sync_copy(orow_v, out_hbm.at[(b * _C + ch) * _FC + c])
            return carry

        lax.fori_loop(0, _C, ch_body, 0)

    return body(imgrows, rid, xu, xl, wuu, wul, wlu, wll)


def kernel(images, feature_map):
    f32 = jnp.float32
    fm32 = feature_map.reshape(_NBC, _Y, _X)
    gxr = jnp.linspace(-1.0, 1.0, _X, dtype=f32).reshape(1, _X)
    gyc = jnp.linspace(-1.0, 1.0, _Y, dtype=f32).reshape(_Y, 1)
    g32r = jnp.linspace(-1.0, 1.0, _OX, dtype=f32).reshape(1, _OX)
    g32c = jnp.linspace(-1.0, 1.0, _OY, dtype=f32).reshape(_OY, 1)

    (mx, my, sc, rid, xu, xl, wuu, wul, wlu, wll) = _tc_stats(
        fm32, gxr, gyc, g32r, g32c)

    imgrows = images.reshape(_B * _C * _Y, _X)
    out2d = _sc_bilinear(imgrows, rid, xu.reshape(_NBC, _OX),
                         xl.reshape(_NBC, _OX), wuu, wul, wlu, wll)

    out = out2d.reshape(_B, _C, _FC, _OY, _OX)
    mean_x = mx[:, 0, 0].reshape(_B, _FC)
    mean_y = my[:, 0, 0].reshape(_B, _FC)
    scale = sc[:, 0, 0].reshape(_B, _FC)
    return (out, mean_x, mean_y, scale)


# speedup vs baseline: 1.1130x; 1.1130x over previous
import functools

import jax
import jax.numpy as jnp
from jax import lax
from jax.experimental import pallas as pl
from jax.experimental.pallas import tpu as pltpu
from jax.experimental.pallas import tpu_sc as plsc

_B, _C, _Y, _X = 8, 64, 224, 224
_FC = 4
_NBC = _B * _FC
_OY, _OX = 32, 32


def _stats_body(fm_ref, gxr_ref, gyc_ref, g32r_ref, g32c_ref,
                mx_ref, my_ref, sc_ref, rid_ref, xu_ref, xl_ref,
                wuu_ref, wul_ref, wlu_ref, wll_ref):
    bc = pl.program_id(0)
    f = fm_ref[0]
    m = jnp.max(f)
    e = jnp.exp(f - m)
    col = jnp.sum(e, axis=0, keepdims=True)
    row = jnp.sum(e, axis=1, keepdims=True)
    s_tot = jnp.sum(col)
    gx = gxr_ref[...]
    gy = gyc_ref[...]
    mean_x = jnp.sum(col * gx) / s_tot
    mean_y = jnp.sum(row * gy) / s_tot
    scale = (jnp.sum(col * jnp.abs(gx - mean_x))
             + jnp.sum(row * jnp.abs(gy - mean_y))) / s_tot

    g32r = g32r_ref[...]
    g32c = g32c_ref[...]
    x_raw = ((g32r * scale + mean_x) + 1.0) * (_X / 2.0)
    xu = jnp.clip(jnp.ceil(x_raw), 0.0, _X - 1.0)
    xl = jnp.clip(jnp.floor(x_raw), 0.0, _X - 1.0)
    wxu = x_raw - xl
    wxl = xu - x_raw
    y_rawr = ((g32r * scale + mean_y) + 1.0) * (_Y / 2.0)
    yur = jnp.clip(jnp.ceil(y_rawr), 0.0, _Y - 1.0)
    ylr = jnp.clip(jnp.floor(y_rawr), 0.0, _Y - 1.0)
    y_rawc = ((g32c * scale + mean_y) + 1.0) * (_Y / 2.0)
    yuc = jnp.clip(jnp.ceil(y_rawc), 0.0, _Y - 1.0)
    ylc = jnp.clip(jnp.floor(y_rawc), 0.0, _Y - 1.0)
    wyu = y_rawc - ylc
    wyl = yuc - y_rawc

    mx_ref[...] = jnp.full((1, 1, 32), mean_x, jnp.float32)
    my_ref[...] = jnp.full((1, 1, 32), mean_y, jnp.float32)
    sc_ref[...] = jnp.full((1, 1, 32), scale, jnp.float32)

    b = bc // _FC
    ybase = jnp.concatenate([yur, ylr], axis=1).astype(jnp.int32)
    chv = lax.broadcasted_iota(jnp.int32, (_C, 1), 0)
    rid = (b * _C + chv) * _Y + ybase
    rid_ref[...] = rid.reshape(1, _C, 2 * _OY)

    xu_ref[...] = xu.astype(jnp.int32).reshape(1, 1, _OX)
    xl_ref[...] = xl.astype(jnp.int32).reshape(1, 1, _OX)
    wuu_ref[...] = (wyu * wxu).reshape(1, _OY, _OX)
    wul_ref[...] = (wyu * wxl).reshape(1, _OY, _OX)
    wlu_ref[...] = (wyl * wxu).reshape(1, _OY, _OX)
    wll_ref[...] = (wyl * wxl).reshape(1, _OY, _OX)


def _tc_stats(fm32, gxr, gyc, g32r, g32c):
    n = _NBC
    f32 = jnp.float32
    i32 = jnp.int32
    outs = [
        jax.ShapeDtypeStruct((n, 1, 32), f32),
        jax.ShapeDtypeStruct((n, 1, 32), f32),
        jax.ShapeDtypeStruct((n, 1, 32), f32),
        jax.ShapeDtypeStruct((n, _C, 2 * _OY), i32),
        jax.ShapeDtypeStruct((n, 1, _OX), i32),
        jax.ShapeDtypeStruct((n, 1, _OX), i32),
        jax.ShapeDtypeStruct((n, _OY, _OX), f32),
        jax.ShapeDtypeStruct((n, _OY, _OX), f32),
        jax.ShapeDtypeStruct((n, _OY, _OX), f32),
        jax.ShapeDtypeStruct((n, _OY, _OX), f32),
    ]
    full = lambda shp: pl.BlockSpec(shp, lambda i: (0, 0))
    blk = lambda shp: pl.BlockSpec(shp, lambda i: (i, 0, 0))
    return pl.pallas_call(
        _stats_body,
        grid=(n,),
        in_specs=[
            pl.BlockSpec((1, _Y, _X), lambda i: (i, 0, 0)),
            full((1, _X)), full((_Y, 1)), full((1, _OX)), full((_OY, 1)),
        ],
        out_specs=[
            blk((1, 1, 32)), blk((1, 1, 32)), blk((1, 1, 32)),
            blk((1, _C, 2 * _OY)), blk((1, 1, _OX)), blk((1, 1, _OX)),
            blk((1, _OY, _OX)), blk((1, _OY, _OX)),
            blk((1, _OY, _OX)), blk((1, _OY, _OX)),
        ],
        out_shape=outs,
    )(fm32, gxr, gyc, g32r, g32c)


def _sc_bilinear(imgrows, rid, xu, xl, wuu, wul, wlu, wll):
    mesh = plsc.VectorSubcoreMesh(core_axis_name="c", subcore_axis_name="s")

    @functools.partial(
        pl.kernel,
        mesh=mesh,
        compiler_params=pltpu.CompilerParams(use_tc_tiling_on_sc=False,
                                             needs_layout_passes=False),
        out_type=jax.ShapeDtypeStruct((_B * _C * _FC, _OY * _OX), jnp.float32),
        scratch_types=[
            pltpu.VMEM((_C, 2 * _OY), jnp.int32),
            pltpu.VMEM((2 * _OY, _X), jnp.float32),
            pltpu.VMEM((2 * _OY, _X), jnp.float32),
            pltpu.VMEM((_OX,), jnp.int32),
            pltpu.VMEM((_OX,), jnp.int32),
            pltpu.VMEM((_OY, _OX), jnp.float32),
            pltpu.VMEM((_OY, _OX), jnp.float32),
            pltpu.VMEM((_OY, _OX), jnp.float32),
            pltpu.VMEM((_OY, _OX), jnp.float32),
            pltpu.VMEM((_OY * _OX,), jnp.float32),
            pltpu.VMEM((_OY * _OX,), jnp.float32),
            pltpu.SemaphoreType.DMA,
            pltpu.SemaphoreType.DMA,
            pltpu.SemaphoreType.DMA,
            pltpu.SemaphoreType.DMA,
        ],
    )
    def body(img_hbm, rid_hbm, xu_hbm, xl_hbm, wuu_hbm, wul_hbm, wlu_hbm,
             wll_hbm, out_hbm, rid_v, rows0_v, rows1_v, xu_v, xl_v, wuu_v,
             wul_v, wlu_v, wll_v, orow0_v, orow1_v, gsem0, gsem1, osem0,
             osem1):
        wid = lax.axis_index("s") * 2 + lax.axis_index("c")
        b = wid // _FC
        c = wid % _FC
        obase = (b * _C) * _FC + c
        pltpu.sync_copy(rid_hbm.at[wid], rid_v)
        pltpu.sync_copy(xu_hbm.at[wid], xu_v)
        pltpu.sync_copy(xl_hbm.at[wid], xl_v)
        pltpu.sync_copy(wuu_hbm.at[wid], wuu_v)
        pltpu.sync_copy(wul_hbm.at[wid], wul_v)
        pltpu.sync_copy(wlu_hbm.at[wid], wlu_v)
        pltpu.sync_copy(wll_hbm.at[wid], wll_v)

        def combine(rows_v, orow_v):
            for i in range(_OY):
                for h in range(2):
                    sl = pl.ds(h * 16, 16)
                    xu16 = xu_v[sl]
                    xl16 = xl_v[sl]
                    yu16 = jnp.full((16,), i, jnp.int32)
                    yl16 = jnp.full((16,), _OY + i, jnp.int32)
                    puu = plsc.load_gather(rows_v, [yu16, xu16])
                    pul = plsc.load_gather(rows_v, [yu16, xl16])
                    plu = plsc.load_gather(rows_v, [yl16, xu16])
                    pll = plsc.load_gather(rows_v, [yl16, xl16])
                    o = (wuu_v[i, sl] * puu + wul_v[i, sl] * pul
                         + wlu_v[i, sl] * plu + wll_v[i, sl] * pll)
                    orow_v[pl.ds(i * _OX + h * 16, 16)] = o

        def wait_gather(rows_v, gsem):
            pltpu.make_async_copy(img_hbm.at[rid_v.at[0]], rows_v, gsem).wait()

        def drain_out(orow_v, osem):
            pltpu.make_async_copy(out_hbm.at[0], orow_v, osem).wait()

        pltpu.async_copy(img_hbm.at[rid_v.at[0]], rows0_v, gsem0)

        def step(k, carry):
            ch0 = 2 * k
            pltpu.async_copy(img_hbm.at[rid_v.at[ch0 + 1]], rows1_v, gsem1)
            wait_gather(rows0_v, gsem0)

            @pl.when(k > 0)
            def _():
                drain_out(orow0_v, osem0)
            combine(rows0_v, orow0_v)
            pltpu.async_copy(orow0_v, out_hbm.at[obase + ch0 * _FC], osem0)

            @pl.when(k < _C // 2 - 1)
            def _():
                pltpu.async_copy(img_hbm.at[rid_v.at[ch0 + 2]], rows0_v,
                                 gsem0)
            wait_gather(rows1_v, gsem1)

            @pl.when(k > 0)
            def _():
                drain_out(orow1_v, osem1)
            combine(rows1_v, orow1_v)
            pltpu.async_copy(orow1_v, out_hbm.at[obase + (ch0 + 1) * _FC],
                             osem1)
            return carry

        lax.fori_loop(0, _C // 2, step, 0)
        drain_out(orow0_v, osem0)
        drain_out(orow1_v, osem1)

    return body(imgrows, rid, xu, xl, wuu, wul, wlu, wll)


def kernel(images, feature_map):
    f32 = jnp.float32
    fm32 = feature_map.reshape(_NBC, _Y, _X)
    gxr = jnp.linspace(-1.0, 1.0, _X, dtype=f32).reshape(1, _X)
    gyc = jnp.linspace(-1.0, 1.0, _Y, dtype=f32).reshape(_Y, 1)
    g32r = jnp.linspace(-1.0, 1.0, _OX, dtype=f32).reshape(1, _OX)
    g32c = jnp.linspace(-1.0, 1.0, _OY, dtype=f32).reshape(_OY, 1)

    (mx, my, sc, rid, xu, xl, wuu, wul, wlu, wll) = _tc_stats(
        fm32, gxr, gyc, g32r, g32c)

    imgrows = images.reshape(_B * _C * _Y, _X)
    out2d = _sc_bilinear(imgrows, rid, xu.reshape(_NBC, _OX),
                         xl.reshape(_NBC, _OX), wuu, wul, wlu, wll)

    out = out2d.reshape(_B, _C, _FC, _OY, _OX)
    mean_x = mx[:, 0, 0].reshape(_B, _FC)
    mean_y = my[:, 0, 0].reshape(_B, _FC)
    scale = sc[:, 0, 0].reshape(_B, _FC)
    return (out, mean_x, mean_y, scale)
